# Initial kernel scaffold; baseline (speedup 1.0000x reference)
import functools

import jax
import jax.numpy as jnp
from jax import lax
from jax.experimental import pallas as pl
from jax.experimental.pallas import tpu as pltpu
from jax.experimental.pallas import tpu_sc as plsc

D = 64
B = 16384
NNEG = 20
NC = 2
NS = 16
NW = NC * NS
BPW = B // NW
CHUNK = 512


def _sc_gather(u_weight, v_weight, pos_u, pos_v, neg_v_flat):
    mesh = plsc.VectorSubcoreMesh(core_axis_name="c", subcore_axis_name="s")

    @functools.partial(
        pl.kernel,
        out_type=[
            jax.ShapeDtypeStruct((B, D), jnp.float32),
            jax.ShapeDtypeStruct((B, D), jnp.float32),
            jax.ShapeDtypeStruct((B * NNEG, D), jnp.float32),
        ],
        mesh=mesh,
        scratch_types=[
            pltpu.VMEM((CHUNK,), jnp.int32),
            pltpu.VMEM((CHUNK, D), jnp.float32),
        ],
    )
    def gather_kernel(u_hbm, v_hbm, pu_hbm, pv_hbm, nv_hbm,
                      eu_hbm, ev_hbm, en_hbm, idx_v, rows_v):
        wid = lax.axis_index("s") * NC + lax.axis_index("c")

        def stream(idx_hbm, table_hbm, out_hbm, base, nrows):
            @pl.loop(0, nrows // CHUNK)
            def _(i):
                off = base + i * CHUNK
                pltpu.sync_copy(idx_hbm.at[pl.ds(off, CHUNK)], idx_v)
                pltpu.sync_copy(table_hbm.at[idx_v], rows_v)
                pltpu.sync_copy(rows_v, out_hbm.at[pl.ds(off, CHUNK)])

        stream(pu_hbm, u_hbm, eu_hbm, wid * BPW, BPW)
        stream(pv_hbm, v_hbm, ev_hbm, wid * BPW, BPW)
        stream(nv_hbm, v_hbm, en_hbm, wid * BPW * NNEG, BPW * NNEG)

    return gather_kernel(u_weight, v_weight, pos_u, pos_v, neg_v_flat)


_TC_BLK = 512


def _tc_body(eu_ref, ev_ref, en_ref, out_ref):
    u = eu_ref[...]
    v = ev_ref[...]
    neg = en_ref[...].reshape(_TC_BLK, NNEG, D)

    def logsig(x):
        return jnp.minimum(x, 0.0) - jnp.log1p(jnp.exp(-jnp.abs(x)))

    pos_score = jnp.sum(u * v, axis=1)
    neg_score = jnp.sum(neg * u[:, None, :], axis=2)
    total = jnp.sum(logsig(pos_score)) + jnp.sum(logsig(-neg_score))

    @pl.when(pl.program_id(0) == 0)
    def _():
        out_ref[0, 0] = 0.0

    out_ref[0, 0] += -total


def _tc_loss(emb_u, emb_v, neg_rows):
    out = pl.pallas_call(
        _tc_body,
        grid=(B // _TC_BLK,),
        in_specs=[
            pl.BlockSpec((_TC_BLK, D), lambda i: (i, 0)),
            pl.BlockSpec((_TC_BLK, D), lambda i: (i, 0)),
            pl.BlockSpec((_TC_BLK * NNEG, D), lambda i: (i, 0)),
        ],
        out_specs=pl.BlockSpec((1, 1), lambda i: (0, 0)),
        out_shape=jax.ShapeDtypeStruct((1, 1), jnp.float32),
    )(emb_u, emb_v, neg_rows)
    return out[0, 0]


def kernel(u_weight, v_weight, pos_u, pos_v, neg_v):
    neg_flat = neg_v.reshape(B * NNEG)
    emb_u, emb_v, neg_rows = _sc_gather(
        u_weight, v_weight,
        pos_u.astype(jnp.int32), pos_v.astype(jnp.int32),
        neg_flat.astype(jnp.int32))
    return _tc_loss(emb_u, emb_v, neg_rows)


# speedup vs baseline: 4.2128x; 4.2128x over previous
import functools

import jax
import jax.numpy as jnp
from jax import lax
from jax.experimental import pallas as pl
from jax.experimental.pallas import tpu as pltpu
from jax.experimental.pallas import tpu_sc as plsc

D = 64
B = 16384
NNEG = 20
NC = 2
NS = 16
NW = NC * NS
BPW = B // NW
CHUNK = 512


def _sc_gather(u_weight, v_weight, pos_u, pos_v, neg_v_flat):
    mesh = plsc.VectorSubcoreMesh(core_axis_name="c", subcore_axis_name="s")

    @functools.partial(
        pl.kernel,
        out_type=[
            jax.ShapeDtypeStruct((B, D), jnp.float32),
            jax.ShapeDtypeStruct((B, D), jnp.float32),
            jax.ShapeDtypeStruct((B * NNEG, D), jnp.float32),
        ],
        mesh=mesh,
        compiler_params=pltpu.CompilerParams(use_tc_tiling_on_sc=False),
        scratch_types=[
            pltpu.VMEM((CHUNK,), jnp.int32),
            pltpu.VMEM((CHUNK, D), jnp.float32),
        ],
    )
    def gather_kernel(u_hbm, v_hbm, pu_hbm, pv_hbm, nv_hbm,
                      eu_hbm, ev_hbm, en_hbm, idx_v, rows_v):
        wid = lax.axis_index("s") * NC + lax.axis_index("c")

        def stream(idx_hbm, table_hbm, out_hbm, base, nrows):
            @pl.loop(0, nrows // CHUNK)
            def _(i):
                off = base + i * CHUNK
                pltpu.sync_copy(idx_hbm.at[pl.ds(off, CHUNK)], idx_v)
                pltpu.sync_copy(table_hbm.at[idx_v], rows_v)
                pltpu.sync_copy(rows_v, out_hbm.at[pl.ds(off, CHUNK)])

        stream(pu_hbm, u_hbm, eu_hbm, wid * BPW, BPW)
        stream(pv_hbm, v_hbm, ev_hbm, wid * BPW, BPW)
        stream(nv_hbm, v_hbm, en_hbm, wid * BPW * NNEG, BPW * NNEG)

    return gather_kernel(u_weight, v_weight, pos_u, pos_v, neg_v_flat)


_TC_BLK = 512


def _tc_body(eu_ref, ev_ref, en_ref, out_ref):
    u = eu_ref[...]
    v = ev_ref[...]
    neg = en_ref[...].reshape(_TC_BLK, NNEG, D)

    def logsig(x):
        return jnp.minimum(x, 0.0) - jnp.log1p(jnp.exp(-jnp.abs(x)))

    pos_score = jnp.sum(u * v, axis=1)
    neg_score = jnp.sum(neg * u[:, None, :], axis=2)
    total = jnp.sum(logsig(pos_score)) + jnp.sum(logsig(-neg_score))

    @pl.when(pl.program_id(0) == 0)
    def _():
        out_ref[...] = jnp.zeros((1, 1), jnp.float32)

    out_ref[...] += jnp.full((1, 1), -total, jnp.float32)


def _tc_loss(emb_u, emb_v, neg_rows):
    out = pl.pallas_call(
        _tc_body,
        grid=(B // _TC_BLK,),
        in_specs=[
            pl.BlockSpec((_TC_BLK, D), lambda i: (i, 0)),
            pl.BlockSpec((_TC_BLK, D), lambda i: (i, 0)),
            pl.BlockSpec((_TC_BLK * NNEG, D), lambda i: (i, 0)),
        ],
        out_specs=pl.BlockSpec((1, 1), lambda i: (0, 0)),
        out_shape=jax.ShapeDtypeStruct((1, 1), jnp.float32),
    )(emb_u, emb_v, neg_rows)
    return out[0, 0]


def kernel(u_weight, v_weight, pos_u, pos_v, neg_v):
    neg_flat = neg_v.reshape(B * NNEG)
    emb_u, emb_v, neg_rows = _sc_gather(
        u_weight, v_weight,
        pos_u.astype(jnp.int32), pos_v.astype(jnp.int32),
        neg_flat.astype(jnp.int32))
    return _tc_loss(emb_u, emb_v, neg_rows)
